# baseline (speedup 1.0000x reference)
import functools

import jax
import jax.numpy as jnp
from jax import lax
from jax.experimental import pallas as pl
from jax.experimental.pallas import tpu as pltpu
from jax.experimental.pallas import tpu_sc as plsc

NUM_USERS = 1000000
NUM_ITEMS = 1000000
EMBED_DIM = 64
BATCH = 16384

NC = 2
NS = 16
L = 16
NW = NC * NS
BPW = BATCH // NW
RB = BPW // L

_mesh = plsc.VectorSubcoreMesh(
    core_axis_name="c", subcore_axis_name="s", num_cores=NC, num_subcores=NS
)


@functools.partial(
    pl.kernel,
    out_type=jax.ShapeDtypeStruct((BATCH,), jnp.float32),
    mesh=_mesh,
    scratch_types=[
        pltpu.VMEM((BPW,), jnp.int32),
        pltpu.VMEM((BPW,), jnp.int32),
        pltpu.VMEM((BPW, EMBED_DIM), jnp.float32),
        pltpu.VMEM((BPW, EMBED_DIM), jnp.float32),
        pltpu.VMEM((BPW,), jnp.float32),
        pltpu.SemaphoreType.DMA,
        pltpu.SemaphoreType.DMA,
    ],
    compiler_params=pltpu.CompilerParams(
        needs_layout_passes=False, use_tc_tiling_on_sc=False),
)
def _sc_dot(uid_hbm, iid_hbm, ut_hbm, it_hbm, out_hbm,
            uidx_v, iidx_v, urows_v, irows_v, out_v, sem_u, sem_i):
    wid = lax.axis_index("s") * NC + lax.axis_index("c")
    base = wid * BPW
    pltpu.sync_copy(uid_hbm.at[pl.ds(base, BPW)], uidx_v)
    pltpu.sync_copy(iid_hbm.at[pl.ds(base, BPW)], iidx_v)
    cu = pltpu.async_copy(ut_hbm.at[uidx_v], urows_v, sem_u)
    ci = pltpu.async_copy(it_hbm.at[iidx_v], irows_v, sem_i)
    cu.wait()
    ci.wait()

    lanes = lax.iota(jnp.int32, L)

    def rb_body(rb, carry):
        rows = rb * L + lanes
        acc = jnp.zeros((L,), jnp.float32)
        for c in range(EMBED_DIM):
            col = jnp.full((L,), c, jnp.int32)
            acc = acc + plsc.load_gather(urows_v, [rows, col]) * \
                plsc.load_gather(irows_v, [rows, col])
        out_v[pl.ds(rb * L, L)] = acc
        return carry

    lax.fori_loop(0, RB, rb_body, 0)
    pltpu.sync_copy(out_v, out_hbm.at[pl.ds(base, BPW)])


def kernel(user_ids, item_ids, user_table, item_table):
    return _sc_dot(user_ids.astype(jnp.int32), item_ids.astype(jnp.int32),
                   user_table, item_table)
